# Initial kernel scaffold; baseline (speedup 1.0000x reference)
import jax
import jax.numpy as jnp
from jax.experimental import pallas as pl


def kernel(x, W_in, W_out):
    raise NotImplementedError("write your pallas kernel here")


# speedup vs baseline: 4.9531x; 4.9531x over previous
import functools
import math

import jax
import jax.numpy as jnp
from jax.experimental import pallas as pl
from jax.experimental.pallas import tpu as pltpu

_DEPTH = 11
_LEVELS = _DEPTH + 1
_NN = 2 ** (_DEPTH + 1) - 1
_LANE = 128


def _gelu_exact(s):
    return 0.5 * s * (1.0 + jax.lax.erf(s * (1.0 / math.sqrt(2.0))))


def _fff_block_kernel(x_ref, w_in_ref, w_out_ref, out_ref, m_ref):
    bt = x_ref.shape[0]
    x = x_ref[...]
    a = jax.lax.dot_general(
        x, w_in_ref[...],
        dimension_numbers=(((1,), (1,)), ((), ())),
        preferred_element_type=jnp.float32,
        precision=jax.lax.Precision.HIGHEST,
    )
    m_ref[...] = jnp.zeros_like(m_ref)
    node = jnp.zeros((bt, 1), jnp.int32)
    for lvl in range(_LEVELS):
        lo = (1 << lvl) - 1
        width = 1 << lvl
        c0 = (lo // _LANE) * _LANE
        c1 = min(_NN, ((lo + width + _LANE - 1) // _LANE) * _LANE)
        iota = jax.lax.broadcasted_iota(jnp.int32, (bt, c1 - c0), 1) + c0
        mask = iota == node
        a_sl = jax.lax.slice(a, (0, c0), (bt, c1))
        s = jnp.sum(jnp.where(mask, a_sl, 0.0), axis=1, keepdims=True)
        g = _gelu_exact(s)
        m_ref[:, c0:c1] += jnp.where(mask, g, 0.0)
        node = node * 2 + 1 + (s >= 0.0).astype(jnp.int32)
    out_ref[...] = jax.lax.dot_general(
        m_ref[...], w_out_ref[...],
        dimension_numbers=(((1,), (1,)), ((), ())),
        preferred_element_type=jnp.float32,
    )


def kernel(x, W_in, W_out):
    b, s, d = x.shape
    n_tok = b * s
    bt = 256
    grid = n_tok // bt
    x_flat = x.reshape(n_tok, d)
    out = pl.pallas_call(
        _fff_block_kernel,
        grid=(grid,),
        in_specs=[
            pl.BlockSpec((bt, d), lambda i: (i, 0)),
            pl.BlockSpec((_NN, d), lambda i: (0, 0)),
            pl.BlockSpec((d, _NN), lambda i: (0, 0)),
        ],
        out_specs=pl.BlockSpec((bt, d), lambda i: (i, 0)),
        out_shape=jax.ShapeDtypeStruct((n_tok, d), jnp.float32),
        scratch_shapes=[pltpu.VMEM((bt, _NN), jnp.float32)],
    )(x_flat, W_in, W_out)
    return out.reshape(b, s, d)


# speedup vs baseline: 7.4453x; 1.5032x over previous
import functools
import math

import jax
import jax.numpy as jnp
from jax.experimental import pallas as pl
from jax.experimental.pallas import tpu as pltpu

_DEPTH = 11
_LEVELS = _DEPTH + 1
_NN = 2 ** (_DEPTH + 1) - 1
_LANE = 128


def _gelu_exact(s):
    return 0.5 * s * (1.0 + jax.lax.erf(s * (1.0 / math.sqrt(2.0))))


def _dotT(lhs, rhs):
    return jax.lax.dot_general(
        lhs, rhs,
        dimension_numbers=(((1,), (1,)), ((), ())),
        preferred_element_type=jnp.float32,
    )


def _fff_block_kernel(x_ref, w_in_hi_ref, w_in_lo_ref, w_out_ref, out_ref,
                      m_ref, a_ref):
    bt = x_ref.shape[0]
    x = x_ref[...]
    x_hi = x.astype(jnp.bfloat16)
    x_lo = (x - x_hi.astype(jnp.float32)).astype(jnp.bfloat16)
    w_hi = w_in_hi_ref[...]
    w_lo = w_in_lo_ref[...]
    a_ref[...] = _dotT(x_hi, w_hi) + _dotT(x_hi, w_lo) + _dotT(x_lo, w_hi)
    m_ref[...] = jnp.zeros_like(m_ref)
    node = jnp.zeros((bt, 1), jnp.int32)
    for lvl in range(_LEVELS):
        lo = (1 << lvl) - 1
        width = 1 << lvl
        c0 = (lo // _LANE) * _LANE
        c1 = min(_NN, ((lo + width + _LANE - 1) // _LANE) * _LANE)
        iota = jax.lax.broadcasted_iota(jnp.int32, (bt, c1 - c0), 1) + c0
        mask = iota == node
        a_sl = a_ref[:, c0:c1]
        s = jnp.sum(jnp.where(mask, a_sl, 0.0), axis=1, keepdims=True)
        g = _gelu_exact(s)
        m_ref[:, c0:c1] += jnp.where(mask, g, 0.0)
        node = node * 2 + 1 + (s >= 0.0).astype(jnp.int32)
    out_ref[...] = jax.lax.dot_general(
        m_ref[...], w_out_ref[...],
        dimension_numbers=(((1,), (1,)), ((), ())),
        preferred_element_type=jnp.float32,
    )


def kernel(x, W_in, W_out):
    b, s, d = x.shape
    n_tok = b * s
    bt = 256
    grid = n_tok // bt
    x_flat = x.reshape(n_tok, d)
    w_in_hi = W_in.astype(jnp.bfloat16)
    w_in_lo = (W_in - jax.lax.optimization_barrier(w_in_hi)
               .astype(jnp.float32)).astype(jnp.bfloat16)
    out = pl.pallas_call(
        _fff_block_kernel,
        grid=(grid,),
        in_specs=[
            pl.BlockSpec((bt, d), lambda i: (i, 0)),
            pl.BlockSpec((_NN, d), lambda i: (0, 0)),
            pl.BlockSpec((_NN, d), lambda i: (0, 0)),
            pl.BlockSpec((d, _NN), lambda i: (0, 0)),
        ],
        out_specs=pl.BlockSpec((bt, d), lambda i: (i, 0)),
        out_shape=jax.ShapeDtypeStruct((n_tok, d), jnp.float32),
        scratch_shapes=[pltpu.VMEM((bt, _NN), jnp.float32),
                        pltpu.VMEM((bt, _NN), jnp.float32)],
    )(x_flat, w_in_hi, w_in_lo, W_out)
    return out.reshape(b, s, d)
